# baseline (speedup 1.0000x reference)
import functools
import math

import jax
import jax.numpy as jnp
from jax import lax
from jax.experimental import pallas as pl
from jax.experimental.pallas import tpu as pltpu
from jax.experimental.pallas import tpu_sc as plsc

N = 50000
E = 800000
G = 128
VOCAB = 10000

NC = 2
NS = 16
L = 16

B = 128
E_PAD = 802816
N_PAD = 50048
X_PAD = 53248
HP = 4

_MESH = plsc.VectorSubcoreMesh(core_axis_name="c", subcore_axis_name="s")
_SC_PARAMS = pltpu.CompilerParams(use_tc_tiling_on_sc=False,
                                  needs_layout_passes=False)


def _iota16():
    return lax.iota(jnp.int32, 16)


def _full16(v):
    return jnp.full((16,), v, jnp.int32)


def _emb_body(emb_hbm, x_hbm, out_hbm, idx, rows, sem):
    cid = lax.axis_index("c")
    sid = lax.axis_index("s")
    wid = sid * NC + cid
    n_chunks = X_PAD // (B * NC * NS)

    def chunk(t, _):
        e0 = (wid * n_chunks + t) * B
        pltpu.sync_copy(x_hbm.at[pl.ds(e0, B)], idx)
        pltpu.async_copy(emb_hbm.at[idx], rows, sem).wait()
        pltpu.sync_copy(rows, out_hbm.at[pl.ds(e0, B)])
        return 0

    lax.fori_loop(0, n_chunks, chunk, 0)


def _emb_lookup(emb, x_pad):
    f = pl.kernel(
        _emb_body,
        out_type=jax.ShapeDtypeStruct((X_PAD, 16), jnp.float32),
        mesh=_MESH,
        compiler_params=_SC_PARAMS,
        scratch_types=[
            pltpu.VMEM((B,), jnp.int32),
            pltpu.VMEM((B, 16), jnp.float32),
            pltpu.SemaphoreType.DMA,
        ],
    )
    return f(emb, x_pad)


def _ex_body(H, C, qs_hbm, k_hbm, src_hbm, dst_hbm, ex_hbm,
             sbig, dbig, qbufs, kbufs, exbufs, semq, semk, semx):
    cid = lax.axis_index("c")
    sid = lax.axis_index("s")
    wid = sid * NC + cid
    n = E_PAD // (B * NC * NS)
    c0 = wid * n

    pltpu.sync_copy(src_hbm.at[pl.ds(c0, n)], sbig)
    pltpu.sync_copy(dst_hbm.at[pl.ds(c0, n)], dbig)

    zero = jnp.zeros((16,), jnp.float32)
    for exbuf in exbufs:
        for g in range(B // 16):
            rows16 = _iota16() + g * 16
            for h in range(H, HP):
                plsc.store_scatter(exbuf, [rows16, _full16(h)], zero)

    HB = B // 2

    def issue_gather(t, p):
        pltpu.async_copy(qs_hbm.at[dbig.at[t, pl.ds(0, HB)]],
                         qbufs[p].at[pl.ds(0, HB)], semq[p])
        pltpu.async_copy(qs_hbm.at[dbig.at[t, pl.ds(HB, HB)]],
                         qbufs[p].at[pl.ds(HB, HB)], semq[p])
        pltpu.async_copy(k_hbm.at[sbig.at[t, pl.ds(0, HB)]],
                         kbufs[p].at[pl.ds(0, HB)], semk[p])
        pltpu.async_copy(k_hbm.at[sbig.at[t, pl.ds(HB, HB)]],
                         kbufs[p].at[pl.ds(HB, HB)], semk[p])

    def wait_gather(t, p):
        pltpu.make_async_copy(qs_hbm.at[dbig.at[t]], qbufs[p], semq[p]).wait()
        pltpu.make_async_copy(k_hbm.at[sbig.at[t]], kbufs[p], semk[p]).wait()

    def sub(t, p):
        @pl.when(t + 1 < n)
        def _():
            issue_gather(t + 1, 1 - p)
        wait_gather(t, p)

        @pl.when(t >= 2)
        def _():
            pltpu.make_async_copy(
                exbufs[p], ex_hbm.at[pl.ds((c0 + t - 2) * B, B)],
                semx[p]).wait()

        for g in range(B // 16):
            rows16 = _iota16() + g * 16
            for h in range(H):
                acc = None
                for c in range(C):
                    col = _full16(h * C + c)
                    qv = plsc.load_gather(qbufs[p], [rows16, col])
                    kv = plsc.load_gather(kbufs[p], [rows16, col])
                    acc = qv * kv if acc is None else acc + qv * kv
                plsc.store_scatter(exbufs[p], [rows16, _full16(h)],
                                   jnp.exp(acc))
        pltpu.async_copy(exbufs[p], ex_hbm.at[pl.ds((c0 + t) * B, B)],
                         semx[p])

    issue_gather(0, 0)

    def body(i, _):
        sub(2 * i, 0)
        sub(2 * i + 1, 1)
        return 0

    lax.fori_loop(0, n // 2, body, 0)
    pltpu.make_async_copy(exbufs[0], ex_hbm.at[pl.ds((c0 + n - 2) * B, B)],
                          semx[0]).wait()
    pltpu.make_async_copy(exbufs[1], ex_hbm.at[pl.ds((c0 + n - 1) * B, B)],
                          semx[1]).wait()


def _edge_ex(qs, k, src2, dst2, H, C):
    HC = H * C
    n = E_PAD // (B * NC * NS)
    f = pl.kernel(
        functools.partial(_ex_body, H, C),
        out_type=jax.ShapeDtypeStruct((E_PAD, HP), jnp.float32),
        mesh=_MESH,
        compiler_params=_SC_PARAMS,
        scratch_types=[
            pltpu.VMEM((n, B), jnp.int32),
            pltpu.VMEM((n, B), jnp.int32),
            [pltpu.VMEM((B, HC), jnp.float32)] * 2,
            [pltpu.VMEM((B, HC), jnp.float32)] * 2,
            [pltpu.VMEM((B, HP), jnp.float32)] * 2,
            [pltpu.SemaphoreType.DMA] * 2,
            [pltpu.SemaphoreType.DMA] * 2,
            [pltpu.SemaphoreType.DMA] * 2,
        ],
    )
    return f(qs, k, src2, dst2)


W = 8


def _scatter_pass(H, C, h_idx, v_hbm, ex_hbm, src_hbm, dst_hbm,
                  acc_out, den_out, zacc_hbm, zden_hbm,
                  sbig, dbig, vbufs, exbufs, wbufs, accsh, densh,
                  semv, seme, semw, semd, do_den):
    sid = lax.axis_index("s")
    rows_per = N_PAD // NS
    r0 = sid * rows_per
    pltpu.sync_copy(zacc_hbm.at[pl.ds(r0, rows_per)],
                    accsh.at[pl.ds(r0, rows_per)])
    if do_den:
        pltpu.sync_copy(zden_hbm.at[pl.ds(r0, rows_per)],
                        densh.at[pl.ds(r0, rows_per)])
    plsc.subcore_barrier()

    n = E_PAD // (B * NS * 2)
    HB = B // 2

    def issue_gather(t, p, c0):
        pltpu.async_copy(v_hbm.at[sbig.at[t, pl.ds(0, HB)]],
                         vbufs[p].at[pl.ds(0, HB)], semv[p])
        pltpu.async_copy(v_hbm.at[sbig.at[t, pl.ds(HB, HB)]],
                         vbufs[p].at[pl.ds(HB, HB)], semv[p])
        pltpu.async_copy(ex_hbm.at[pl.ds((c0 + t) * B, B)], exbufs[p],
                         seme[p])

    def wait_scatter(t, p):
        pltpu.make_async_copy(wbufs[p], accsh.at[dbig.at[t]], semw[p]).wait()
        if do_den:
            pltpu.make_async_copy(exbufs[p], densh.at[dbig.at[t]],
                                  semd[p]).wait()

    def sub(t, p, c0):
        @pl.when(t >= 1)
        def _():
            wait_scatter(t - 1, 1 - p)

        @pl.when(t + 1 < n)
        def _():
            issue_gather(t + 1, 1 - p, c0)

        pltpu.make_async_copy(v_hbm.at[sbig.at[t]], vbufs[p], semv[p]).wait()
        pltpu.make_async_copy(ex_hbm.at[pl.ds((c0 + t) * B, B)], exbufs[p],
                              seme[p]).wait()
        for g in range(B // 16):
            rows16 = _iota16() + g * 16
            exh = plsc.load_gather(exbufs[p], [rows16, _full16(h_idx)])
            for cc in range(W):
                vv = plsc.load_gather(vbufs[p], [rows16, _full16(cc)])
                plsc.store_scatter(wbufs[p], [rows16, _full16(cc)],
                                   vv * exh)
        pltpu.async_copy(wbufs[p], accsh.at[dbig.at[t]], add=True,
                         sem=semw[p])
        if do_den:
            pltpu.async_copy(exbufs[p], densh.at[dbig.at[t]], add=True,
                             sem=semd[p])

    def half_body(half, _):
        c0 = sid * 2 * n + half * n
        pltpu.sync_copy(src_hbm.at[pl.ds(c0, n)], sbig)
        pltpu.sync_copy(dst_hbm.at[pl.ds(c0, n)], dbig)
        issue_gather(0, 0, c0)

        def body(i, _):
            sub(2 * i, 0, c0)
            sub(2 * i + 1, 1, c0)
            return 0

        lax.fori_loop(0, n // 2, body, 0)
        wait_scatter(n - 1, 1)
        return 0

    lax.fori_loop(0, 2, half_body, 0)

    plsc.subcore_barrier()
    pltpu.sync_copy(accsh.at[pl.ds(r0, rows_per)],
                    acc_out.at[pl.ds(r0, rows_per)])
    if do_den:
        pltpu.sync_copy(densh.at[pl.ds(r0, rows_per)],
                        den_out.at[pl.ds(r0, rows_per)])


def _scatter_body(H, C, P, *refs):
    vs = refs[:2 * P]
    ex_hbm, src_hbm, dst_hbm, zacc_hbm, zden_hbm = refs[2 * P:2 * P + 5]
    accs = refs[2 * P + 5:4 * P + 5]
    den_out = refs[4 * P + 5]
    (sbig, dbig, vbufs, exbufs, wbufs, accsh, densh,
     semv, seme, semw, semd) = refs[4 * P + 6:]
    cid = lax.axis_index("c")

    for c in range(NC):
        @pl.when(cid == c)
        def _(c=c):
            for j in range(P):
                g = 2 * j + c
                _scatter_pass(H, C, (g * W) // C, vs[g], ex_hbm,
                              src_hbm, dst_hbm, accs[g], den_out,
                              zacc_hbm, zden_hbm,
                              sbig, dbig, vbufs, exbufs, wbufs,
                              accsh, densh, semv, seme, semw, semd,
                              do_den=(c == 1 and j == 0))


def _edge_scatter(vgroups, ex, src2, dst2, H, C):
    P = len(vgroups) // 2
    n = E_PAD // (B * NS * 2)
    zacc = jnp.zeros((N_PAD, W), jnp.float32)
    zden = jnp.zeros((N_PAD, HP), jnp.float32)
    f = pl.kernel(
        functools.partial(_scatter_body, H, C, P),
        out_type=[jax.ShapeDtypeStruct((N_PAD, W), jnp.float32)] * (2 * P)
        + [jax.ShapeDtypeStruct((N_PAD, HP), jnp.float32)],
        mesh=_MESH,
        compiler_params=_SC_PARAMS,
        scratch_types=[
            pltpu.VMEM((n, B), jnp.int32),
            pltpu.VMEM((n, B), jnp.int32),
            [pltpu.VMEM((B, W), jnp.float32)] * 2,
            [pltpu.VMEM((B, HP), jnp.float32)] * 2,
            [pltpu.VMEM((B, W), jnp.float32)] * 2,
            pltpu.VMEM_SHARED((N_PAD, W), jnp.float32),
            pltpu.VMEM_SHARED((N_PAD, HP), jnp.float32),
            [pltpu.SemaphoreType.DMA] * 2,
            [pltpu.SemaphoreType.DMA] * 2,
            [pltpu.SemaphoreType.DMA] * 2,
            [pltpu.SemaphoreType.DMA] * 2,
        ],
    )
    outs = f(*vgroups, ex, src2, dst2, zacc, zden)
    return outs[:-1], outs[-1]


def _proj_body(C, NG, h_ref, wq, bq, wk, bk, wv, bv, ws, bs,
               qs_ref, k_ref, hws_ref, *v_refs):
    h = h_ref[...]
    dot = lambda a, b: lax.dot_general(a, b, (((1,), (0,)), ((), ())),
                                       preferred_element_type=jnp.float32)
    qs_ref[...] = (dot(h, wq[...]) + bq[...]) * (1.0 / math.sqrt(C))
    k_ref[...] = dot(h, wk[...]) + bk[...]
    v = dot(h, wv[...]) + bv[...]
    for g in range(NG):
        v_refs[g][...] = v[:, g * W:(g + 1) * W]
    hws_ref[...] = dot(h, ws[...]) + bs[...]


def _projections(h, Wq, bq, Wk, bk, Wv, bv, Ws, bs, C):
    IN = h.shape[1]
    HC = Wq.shape[1]
    NG = HC // W
    R = 3128
    grid = N_PAD // R
    row_spec = lambda w: pl.BlockSpec((R, w), lambda i: (i, 0))
    full = lambda a: pl.BlockSpec(a.shape, lambda i: (0, 0))
    out_shapes = (
        [jax.ShapeDtypeStruct((N_PAD, HC), jnp.float32)] * 3
        + [jax.ShapeDtypeStruct((N_PAD, W), jnp.float32)] * NG
    )
    args = (h, Wq, bq[None, :], Wk, bk[None, :], Wv, bv[None, :],
            Ws, bs[None, :])
    outs = pl.pallas_call(
        functools.partial(_proj_body, C, NG),
        grid=(grid,),
        in_specs=[row_spec(IN)] + [full(a) for a in args[1:]],
        out_specs=[row_spec(HC)] * 3 + [row_spec(W)] * NG,
        out_shape=out_shapes,
    )(*args)
    return outs[0], outs[1], outs[2], outs[3:]


def _final_body(H, C, NG, *refs):
    acc_refs = refs[:NG]
    den_ref, hws_ref, out_ref = refs[NG:]
    acc = jnp.concatenate([r[...] for r in acc_refs], axis=1)
    dn = den_ref[...]
    R = acc.shape[0]
    denb = jnp.concatenate(
        [jnp.broadcast_to(dn[:, h:h + 1], (R, C)) for h in range(H)], axis=1)
    out_ref[...] = acc / (denb + 1e-16) + hws_ref[...]


def _finalize(accs, den, hws, H, C):
    HC = H * C
    NG = len(accs)
    R = 3128
    grid = N_PAD // R
    spec = lambda w: pl.BlockSpec((R, w), lambda i: (i, 0))
    return pl.pallas_call(
        functools.partial(_final_body, H, C, NG),
        grid=(grid,),
        in_specs=[spec(W)] * NG + [spec(HP), spec(HC)],
        out_specs=spec(HC),
        out_shape=jax.ShapeDtypeStruct((N_PAD, HC), jnp.float32),
    )(*accs, den, hws)


def _pool_body(h_ref, b_ref, out_ref):
    i = pl.program_id(0)
    h = h_ref[...]
    R = h.shape[0]
    b = b_ref[...].reshape(R)
    oh = (b[:, None] == lax.broadcasted_iota(jnp.int32, (1, G), 1)
          ).astype(jnp.float32)
    hext = jnp.concatenate([h, jnp.ones((R, 1), jnp.float32)], axis=1)
    part = lax.dot_general(oh, hext, (((0,), (0,)), ((), ())),
                           preferred_element_type=jnp.float32)

    @pl.when(i == 0)
    def _():
        out_ref[...] = jnp.zeros_like(out_ref)

    out_ref[...] += part


def _pool(h3, batch3d):
    R = 3128
    grid = N_PAD // R
    D = h3.shape[1]
    return pl.pallas_call(
        _pool_body,
        grid=(grid,),
        in_specs=[pl.BlockSpec((R, D), lambda i: (i, 0)),
                  pl.BlockSpec((1, 1, R), lambda i: (i, 0, 0))],
        out_specs=pl.BlockSpec((G, D + 1), lambda i: (0, 0)),
        out_shape=jax.ShapeDtypeStruct((G, D + 1), jnp.float32),
    )(h3, batch3d)


def _cls_body(gs_ref, demo_ref, wc1, bc1, wc2, bc2, out_ref):
    gs = gs_ref[...]
    D = gs.shape[1] - 1
    gmean = gs[:, :D] / jnp.maximum(gs[:, D:], 1.0)
    feat = jnp.concatenate([gmean, demo_ref[...]], axis=1)
    dot = lambda a, b: lax.dot_general(a, b, (((1,), (0,)), ((), ())),
                                       preferred_element_type=jnp.float32)
    hh = jnp.maximum(dot(feat, wc1[...]) + bc1[...], 0.0)
    out_ref[...] = dot(hh, wc2[...]) + bc2[...]


def _classifier(gs, demo, Wc1, bc1, Wc2, bc2):
    return pl.pallas_call(
        _cls_body,
        out_shape=jax.ShapeDtypeStruct((G, Wc2.shape[1]), jnp.float32),
    )(gs, demo, Wc1, bc1[None, :], Wc2, bc2[None, :])


def _tconv_layer(h, srcp, dstp, Wq, bq, Wk, bk, Wv, bv, Ws, bs, H, C):
    qs, k, hws, vgroups = _projections(h, Wq, bq, Wk, bk, Wv, bv, Ws, bs, C)
    ex = _edge_ex(qs, k, srcp, dstp, H, C)
    accs, den = _edge_scatter(vgroups, ex, srcp, dstp, H, C)
    return _finalize(accs, den, hws, H, C)


def kernel(x, edge_index, batch, demographics, emb,
           Wq1, bq1, Wk1, bk1, Wv1, bv1, Ws1, bs1,
           Wq2, bq2, Wk2, bk2, Wv2, bv2, Ws2, bs2,
           Wq3, bq3, Wk3, bk3, Wv3, bv3, Ws3, bs3,
           Wc1, bc1, Wc2, bc2):
    src = edge_index[0]
    dst = edge_index[1]

    pad_e = E_PAD - E
    srcp = jnp.concatenate(
        [src, (jnp.arange(pad_e, dtype=jnp.int32) * 97) % N]
    ).reshape(E_PAD // B, B)
    dstp = jnp.concatenate(
        [dst, N + (jnp.arange(pad_e, dtype=jnp.int32) % (N_PAD - N))]
    ).reshape(E_PAD // B, B)

    x_pad = jnp.concatenate(
        [x, jnp.arange(X_PAD - N, dtype=jnp.int32) % VOCAB])

    h0 = _emb_lookup(emb, x_pad)[:N_PAD]

    h1 = _tconv_layer(h0, srcp, dstp, Wq1, bq1, Wk1, bk1, Wv1, bv1,
                      Ws1, bs1, 3, 16)
    h2 = _tconv_layer(h1, srcp, dstp, Wq2, bq2, Wk2, bk2, Wv2, bv2,
                      Ws2, bs2, 1, 48)
    h3 = _tconv_layer(h2, srcp, dstp, Wq3, bq3, Wk3, bk3, Wv3, bv3,
                      Ws3, bs3, 1, 32)

    batchp = jnp.concatenate(
        [batch, jnp.full((N_PAD - N,), G, jnp.int32)]).reshape(16, 1, 3128)
    gs = _pool(h3, batchp)
    return _classifier(gs, demographics, Wc1, bc1, Wc2, bc2)


# speedup vs baseline: 1.4458x; 1.4458x over previous
import functools
import math

import jax
import jax.numpy as jnp
from jax import lax
from jax.experimental import pallas as pl
from jax.experimental.pallas import tpu as pltpu
from jax.experimental.pallas import tpu_sc as plsc

N = 50000
E = 800000
G = 128
VOCAB = 10000

NC = 2
NS = 16
L = 16

B = 128
E_PAD = 802816
N_PAD = 50048
X_PAD = 53248
HP = 4

_MESH = plsc.VectorSubcoreMesh(core_axis_name="c", subcore_axis_name="s")
_SC_PARAMS = pltpu.CompilerParams(use_tc_tiling_on_sc=False,
                                  needs_layout_passes=False)


def _iota16():
    return lax.iota(jnp.int32, 16)


def _full16(v):
    return jnp.full((16,), v, jnp.int32)


def _emb_body(emb_hbm, x_hbm, out_hbm, idx, rows, sem):
    cid = lax.axis_index("c")
    sid = lax.axis_index("s")
    wid = sid * NC + cid
    n_chunks = X_PAD // (B * NC * NS)

    def chunk(t, _):
        e0 = (wid * n_chunks + t) * B
        pltpu.sync_copy(x_hbm.at[pl.ds(e0, B)], idx)
        pltpu.async_copy(emb_hbm.at[idx], rows, sem).wait()
        pltpu.sync_copy(rows, out_hbm.at[pl.ds(e0, B)])
        return 0

    lax.fori_loop(0, n_chunks, chunk, 0)


def _emb_lookup(emb, x_pad):
    f = pl.kernel(
        _emb_body,
        out_type=jax.ShapeDtypeStruct((X_PAD, 16), jnp.float32),
        mesh=_MESH,
        compiler_params=_SC_PARAMS,
        scratch_types=[
            pltpu.VMEM((B,), jnp.int32),
            pltpu.VMEM((B, 16), jnp.float32),
            pltpu.SemaphoreType.DMA,
        ],
    )
    return f(emb, x_pad)


def _ex_body(H, C, qs_hbm, k_hbm, src_hbm, dst_hbm, ex_hbm,
             sbig, dbig, qbufs, kbufs, exbufs, semq, semk, semx):
    cid = lax.axis_index("c")
    sid = lax.axis_index("s")
    wid = sid * NC + cid
    n = E_PAD // (B * NC * NS)
    c0 = wid * n

    pltpu.sync_copy(src_hbm.at[pl.ds(c0, n)], sbig)
    pltpu.sync_copy(dst_hbm.at[pl.ds(c0, n)], dbig)

    zero = jnp.zeros((16,), jnp.float32)
    for exbuf in exbufs:
        for g in range(B // 16):
            rows16 = _iota16() + g * 16
            for h in range(H, HP):
                plsc.store_scatter(exbuf, [rows16, _full16(h)], zero)

    HB = B // 2

    def issue_gather(t, p):
        pltpu.async_copy(qs_hbm.at[dbig.at[t]], qbufs[p], semq[p])
        pltpu.async_copy(k_hbm.at[sbig.at[t]], kbufs[p], semk[p])

    def wait_gather(t, p):
        pltpu.make_async_copy(qs_hbm.at[dbig.at[t]], qbufs[p], semq[p]).wait()
        pltpu.make_async_copy(k_hbm.at[sbig.at[t]], kbufs[p], semk[p]).wait()

    def sub(t, p):
        @pl.when(t + 1 < n)
        def _():
            issue_gather(t + 1, 1 - p)
        wait_gather(t, p)

        @pl.when(t >= 2)
        def _():
            pltpu.make_async_copy(
                exbufs[p], ex_hbm.at[pl.ds((c0 + t - 2) * B, B)],
                semx[p]).wait()

        perm = [jnp.bitwise_and(_iota16() + s, 15) for s in range(16)]

        def group(g, _):
            rows16 = _iota16() + g * 16
            for h in range(H):
                acc = None
                for b in range(C // 16):
                    base = h * C + 16 * b
                    for s in range(16):
                        col = perm[s] + base
                        qv = plsc.load_gather(qbufs[p], [rows16, col])
                        kv = plsc.load_gather(kbufs[p], [rows16, col])
                        acc = qv * kv if acc is None else acc + qv * kv
                plsc.store_scatter(exbufs[p], [rows16, _full16(h)],
                                   jnp.exp(acc))
            return 0

        lax.fori_loop(0, B // 16, group, 0)
        pltpu.async_copy(exbufs[p], ex_hbm.at[pl.ds((c0 + t) * B, B)],
                         semx[p])

    issue_gather(0, 0)

    def body(i, _):
        sub(2 * i, 0)
        sub(2 * i + 1, 1)
        return 0

    lax.fori_loop(0, n // 2, body, 0)
    pltpu.make_async_copy(exbufs[0], ex_hbm.at[pl.ds((c0 + n - 2) * B, B)],
                          semx[0]).wait()
    pltpu.make_async_copy(exbufs[1], ex_hbm.at[pl.ds((c0 + n - 1) * B, B)],
                          semx[1]).wait()


def _edge_ex(qs, k, src2, dst2, H, C):
    HC = H * C
    n = E_PAD // (B * NC * NS)
    f = pl.kernel(
        functools.partial(_ex_body, H, C),
        out_type=jax.ShapeDtypeStruct((E_PAD, HP), jnp.float32),
        mesh=_MESH,
        compiler_params=_SC_PARAMS,
        scratch_types=[
            pltpu.VMEM((n, B), jnp.int32),
            pltpu.VMEM((n, B), jnp.int32),
            [pltpu.VMEM((B, HC), jnp.float32)] * 2,
            [pltpu.VMEM((B, HC), jnp.float32)] * 2,
            [pltpu.VMEM((B, HP), jnp.float32)] * 2,
            [pltpu.SemaphoreType.DMA] * 2,
            [pltpu.SemaphoreType.DMA] * 2,
            [pltpu.SemaphoreType.DMA] * 2,
        ],
    )
    return f(qs, k, src2, dst2)


W = 8


def _scatter_pass(H, C, h_idx, v_hbm, ex_hbm, src_hbm, dst_hbm,
                  acc_out, den_out, zacc_hbm, zden_hbm,
                  sbig, dbig, vbufs, exbufs, wbufs, accsh, densh,
                  semv, seme, semw, semd, do_den):
    sid = lax.axis_index("s")
    rows_per = N_PAD // NS
    r0 = sid * rows_per
    pltpu.sync_copy(zacc_hbm.at[pl.ds(r0, rows_per)],
                    accsh.at[pl.ds(r0, rows_per)])
    if do_den:
        pltpu.sync_copy(zden_hbm.at[pl.ds(r0, rows_per)],
                        densh.at[pl.ds(r0, rows_per)])
    plsc.subcore_barrier()

    n = E_PAD // (B * NS * 2)
    HB = B // 2

    def issue_gather(t, p, c0):
        pltpu.async_copy(v_hbm.at[sbig.at[t]], vbufs[p], semv[p])
        pltpu.async_copy(ex_hbm.at[pl.ds((c0 + t) * B, B)], exbufs[p],
                         seme[p])

    def wait_scatter(t, p):
        pltpu.make_async_copy(wbufs[p], accsh.at[dbig.at[t]], semw[p]).wait()
        if do_den:
            pltpu.make_async_copy(exbufs[p], densh.at[dbig.at[t]],
                                  semd[p]).wait()

    def sub(t, p, c0):
        @pl.when(t >= 1)
        def _():
            wait_scatter(t - 1, 1 - p)

        @pl.when(t + 1 < n)
        def _():
            issue_gather(t + 1, 1 - p, c0)

        pltpu.make_async_copy(v_hbm.at[sbig.at[t]], vbufs[p], semv[p]).wait()
        pltpu.make_async_copy(ex_hbm.at[pl.ds((c0 + t) * B, B)], exbufs[p],
                              seme[p]).wait()
        perm = [jnp.bitwise_and(_iota16() + s, W - 1) for s in range(W)]
        for g in range(B // 16):
            rows16 = _iota16() + g * 16
            exh = plsc.load_gather(exbufs[p], [rows16, _full16(h_idx)])
            for s in range(W):
                vv = plsc.load_gather(vbufs[p], [rows16, perm[s]])
                plsc.store_scatter(wbufs[p], [rows16, perm[s]], vv * exh)
        pltpu.async_copy(wbufs[p], accsh.at[dbig.at[t]], add=True,
                         sem=semw[p])
        if do_den:
            pltpu.async_copy(exbufs[p], densh.at[dbig.at[t]], add=True,
                             sem=semd[p])

    def half_body(half, _):
        c0 = sid * 2 * n + half * n
        pltpu.sync_copy(src_hbm.at[pl.ds(c0, n)], sbig)
        pltpu.sync_copy(dst_hbm.at[pl.ds(c0, n)], dbig)
        issue_gather(0, 0, c0)

        def body(i, _):
            sub(2 * i, 0, c0)
            sub(2 * i + 1, 1, c0)
            return 0

        lax.fori_loop(0, n // 2, body, 0)
        wait_scatter(n - 1, 1)
        return 0

    lax.fori_loop(0, 2, half_body, 0)

    plsc.subcore_barrier()
    pltpu.sync_copy(accsh.at[pl.ds(r0, rows_per)],
                    acc_out.at[pl.ds(r0, rows_per)])
    if do_den:
        pltpu.sync_copy(densh.at[pl.ds(r0, rows_per)],
                        den_out.at[pl.ds(r0, rows_per)])


def _scatter_body(H, C, P, *refs):
    vs = refs[:2 * P]
    ex_hbm, src_hbm, dst_hbm, zacc_hbm, zden_hbm = refs[2 * P:2 * P + 5]
    accs = refs[2 * P + 5:4 * P + 5]
    den_out = refs[4 * P + 5]
    (sbig, dbig, vbufs, exbufs, wbufs, accsh, densh,
     semv, seme, semw, semd) = refs[4 * P + 6:]
    cid = lax.axis_index("c")

    for c in range(NC):
        @pl.when(cid == c)
        def _(c=c):
            for j in range(P):
                g = 2 * j + c
                _scatter_pass(H, C, (g * W) // C, vs[g], ex_hbm,
                              src_hbm, dst_hbm, accs[g], den_out,
                              zacc_hbm, zden_hbm,
                              sbig, dbig, vbufs, exbufs, wbufs,
                              accsh, densh, semv, seme, semw, semd,
                              do_den=(c == 1 and j == 0))


def _edge_scatter(vgroups, ex, src2, dst2, H, C):
    P = len(vgroups) // 2
    n = E_PAD // (B * NS * 2)
    zacc = jnp.zeros((N_PAD, W), jnp.float32)
    zden = jnp.zeros((N_PAD, HP), jnp.float32)
    f = pl.kernel(
        functools.partial(_scatter_body, H, C, P),
        out_type=[jax.ShapeDtypeStruct((N_PAD, W), jnp.float32)] * (2 * P)
        + [jax.ShapeDtypeStruct((N_PAD, HP), jnp.float32)],
        mesh=_MESH,
        compiler_params=_SC_PARAMS,
        scratch_types=[
            pltpu.VMEM((n, B), jnp.int32),
            pltpu.VMEM((n, B), jnp.int32),
            [pltpu.VMEM((B, W), jnp.float32)] * 2,
            [pltpu.VMEM((B, HP), jnp.float32)] * 2,
            [pltpu.VMEM((B, W), jnp.float32)] * 2,
            pltpu.VMEM_SHARED((N_PAD, W), jnp.float32),
            pltpu.VMEM_SHARED((N_PAD, HP), jnp.float32),
            [pltpu.SemaphoreType.DMA] * 2,
            [pltpu.SemaphoreType.DMA] * 2,
            [pltpu.SemaphoreType.DMA] * 2,
            [pltpu.SemaphoreType.DMA] * 2,
        ],
    )
    outs = f(*vgroups, ex, src2, dst2, zacc, zden)
    return outs[:-1], outs[-1]


def _proj_body(C, NG, h_ref, wq, bq, wk, bk, wv, bv, ws, bs,
               qs_ref, k_ref, hws_ref, *v_refs):
    h = h_ref[...]
    dot = lambda a, b: lax.dot_general(a, b, (((1,), (0,)), ((), ())),
                                       preferred_element_type=jnp.float32)
    qs_ref[...] = (dot(h, wq[...]) + bq[...]) * (1.0 / math.sqrt(C))
    k_ref[...] = dot(h, wk[...]) + bk[...]
    v = dot(h, wv[...]) + bv[...]
    for g in range(NG):
        v_refs[g][...] = v[:, g * W:(g + 1) * W]
    hws_ref[...] = dot(h, ws[...]) + bs[...]


def _projections(h, Wq, bq, Wk, bk, Wv, bv, Ws, bs, C):
    IN = h.shape[1]
    HC = Wq.shape[1]
    NG = HC // W
    R = 3128
    grid = N_PAD // R
    row_spec = lambda w: pl.BlockSpec((R, w), lambda i: (i, 0))
    full = lambda a: pl.BlockSpec(a.shape, lambda i: (0, 0))
    out_shapes = (
        [jax.ShapeDtypeStruct((N_PAD, HC), jnp.float32)] * 3
        + [jax.ShapeDtypeStruct((N_PAD, W), jnp.float32)] * NG
    )
    args = (h, Wq, bq[None, :], Wk, bk[None, :], Wv, bv[None, :],
            Ws, bs[None, :])
    outs = pl.pallas_call(
        functools.partial(_proj_body, C, NG),
        grid=(grid,),
        in_specs=[row_spec(IN)] + [full(a) for a in args[1:]],
        out_specs=[row_spec(HC)] * 3 + [row_spec(W)] * NG,
        out_shape=out_shapes,
    )(*args)
    return outs[0], outs[1], outs[2], outs[3:]


def _final_body(H, C, NG, *refs):
    acc_refs = refs[:NG]
    den_ref, hws_ref, out_ref = refs[NG:]
    acc = jnp.concatenate([r[...] for r in acc_refs], axis=1)
    dn = den_ref[...]
    R = acc.shape[0]
    denb = jnp.concatenate(
        [jnp.broadcast_to(dn[:, h:h + 1], (R, C)) for h in range(H)], axis=1)
    out_ref[...] = acc / (denb + 1e-16) + hws_ref[...]


def _finalize(accs, den, hws, H, C):
    HC = H * C
    NG = len(accs)
    R = 3128
    grid = N_PAD // R
    spec = lambda w: pl.BlockSpec((R, w), lambda i: (i, 0))
    return pl.pallas_call(
        functools.partial(_final_body, H, C, NG),
        grid=(grid,),
        in_specs=[spec(W)] * NG + [spec(HP), spec(HC)],
        out_specs=spec(HC),
        out_shape=jax.ShapeDtypeStruct((N_PAD, HC), jnp.float32),
    )(*accs, den, hws)


def _pool_body(h_ref, b_ref, out_ref):
    i = pl.program_id(0)
    h = h_ref[...]
    R = h.shape[0]
    b = b_ref[...].reshape(R)
    oh = (b[:, None] == lax.broadcasted_iota(jnp.int32, (1, G), 1)
          ).astype(jnp.float32)
    hext = jnp.concatenate([h, jnp.ones((R, 1), jnp.float32)], axis=1)
    part = lax.dot_general(oh, hext, (((0,), (0,)), ((), ())),
                           preferred_element_type=jnp.float32)

    @pl.when(i == 0)
    def _():
        out_ref[...] = jnp.zeros_like(out_ref)

    out_ref[...] += part


def _pool(h3, batch3d):
    R = 3128
    grid = N_PAD // R
    D = h3.shape[1]
    return pl.pallas_call(
        _pool_body,
        grid=(grid,),
        in_specs=[pl.BlockSpec((R, D), lambda i: (i, 0)),
                  pl.BlockSpec((1, 1, R), lambda i: (i, 0, 0))],
        out_specs=pl.BlockSpec((G, D + 1), lambda i: (0, 0)),
        out_shape=jax.ShapeDtypeStruct((G, D + 1), jnp.float32),
    )(h3, batch3d)


def _cls_body(gs_ref, demo_ref, wc1, bc1, wc2, bc2, out_ref):
    gs = gs_ref[...]
    D = gs.shape[1] - 1
    gmean = gs[:, :D] / jnp.maximum(gs[:, D:], 1.0)
    feat = jnp.concatenate([gmean, demo_ref[...]], axis=1)
    dot = lambda a, b: lax.dot_general(a, b, (((1,), (0,)), ((), ())),
                                       preferred_element_type=jnp.float32)
    hh = jnp.maximum(dot(feat, wc1[...]) + bc1[...], 0.0)
    out_ref[...] = dot(hh, wc2[...]) + bc2[...]


def _classifier(gs, demo, Wc1, bc1, Wc2, bc2):
    return pl.pallas_call(
        _cls_body,
        out_shape=jax.ShapeDtypeStruct((G, Wc2.shape[1]), jnp.float32),
    )(gs, demo, Wc1, bc1[None, :], Wc2, bc2[None, :])


def _tconv_layer(h, srcp, dstp, Wq, bq, Wk, bk, Wv, bv, Ws, bs, H, C):
    qs, k, hws, vgroups = _projections(h, Wq, bq, Wk, bk, Wv, bv, Ws, bs, C)
    ex = _edge_ex(qs, k, srcp, dstp, H, C)
    accs, den = _edge_scatter(vgroups, ex, srcp, dstp, H, C)
    return _finalize(accs, den, hws, H, C)


def kernel(x, edge_index, batch, demographics, emb,
           Wq1, bq1, Wk1, bk1, Wv1, bv1, Ws1, bs1,
           Wq2, bq2, Wk2, bk2, Wv2, bv2, Ws2, bs2,
           Wq3, bq3, Wk3, bk3, Wv3, bv3, Ws3, bs3,
           Wc1, bc1, Wc2, bc2):
    src = edge_index[0]
    dst = edge_index[1]

    pad_e = E_PAD - E
    srcp = jnp.concatenate(
        [src, (jnp.arange(pad_e, dtype=jnp.int32) * 97) % N]
    ).reshape(E_PAD // B, B)
    dstp = jnp.concatenate(
        [dst, N + (jnp.arange(pad_e, dtype=jnp.int32) % (N_PAD - N))]
    ).reshape(E_PAD // B, B)

    x_pad = jnp.concatenate(
        [x, jnp.arange(X_PAD - N, dtype=jnp.int32) % VOCAB])

    h0 = _emb_lookup(emb, x_pad)[:N_PAD]

    h1 = _tconv_layer(h0, srcp, dstp, Wq1, bq1, Wk1, bk1, Wv1, bv1,
                      Ws1, bs1, 3, 16)
    h2 = _tconv_layer(h1, srcp, dstp, Wq2, bq2, Wk2, bk2, Wv2, bv2,
                      Ws2, bs2, 1, 48)
    h3 = _tconv_layer(h2, srcp, dstp, Wq3, bq3, Wk3, bk3, Wv3, bv3,
                      Ws3, bs3, 1, 32)

    batchp = jnp.concatenate(
        [batch, jnp.full((N_PAD - N,), G, jnp.int32)]).reshape(16, 1, 3128)
    gs = _pool(h3, batchp)
    return _classifier(gs, demographics, Wc1, bc1, Wc2, bc2)


# speedup vs baseline: 1.4659x; 1.0139x over previous
import functools
import math

import jax
import jax.numpy as jnp
from jax import lax
from jax.experimental import pallas as pl
from jax.experimental.pallas import tpu as pltpu
from jax.experimental.pallas import tpu_sc as plsc

N = 50000
E = 800000
G = 128
VOCAB = 10000

NC = 2
NS = 16
L = 16

B = 128
E_PAD = 802816
N_PAD = 50048
X_PAD = 53248
HP = 4

_MESH = plsc.VectorSubcoreMesh(core_axis_name="c", subcore_axis_name="s")
_SC_PARAMS = pltpu.CompilerParams(use_tc_tiling_on_sc=False,
                                  needs_layout_passes=False)


def _iota16():
    return lax.iota(jnp.int32, 16)


def _full16(v):
    return jnp.full((16,), v, jnp.int32)


def _emb_body(emb_hbm, x_hbm, out_hbm, idx, rows, sem):
    cid = lax.axis_index("c")
    sid = lax.axis_index("s")
    wid = sid * NC + cid
    n_chunks = X_PAD // (B * NC * NS)

    def chunk(t, _):
        gc = wid * n_chunks + t
        e0 = gc * B

        @pl.when(gc < N_PAD // B)
        def _():
            pltpu.sync_copy(x_hbm.at[pl.ds(e0, B)], idx)
            pltpu.async_copy(emb_hbm.at[idx], rows, sem).wait()
            pltpu.sync_copy(rows, out_hbm.at[pl.ds(e0, B)])
        return 0

    lax.fori_loop(0, n_chunks, chunk, 0)


def _emb_lookup(emb, x_pad):
    f = pl.kernel(
        _emb_body,
        out_type=jax.ShapeDtypeStruct((N_PAD, 16), jnp.float32),
        mesh=_MESH,
        compiler_params=_SC_PARAMS,
        scratch_types=[
            pltpu.VMEM((B,), jnp.int32),
            pltpu.VMEM((B, 16), jnp.float32),
            pltpu.SemaphoreType.DMA,
        ],
    )
    return f(emb, x_pad)


def _ex_body(H, C, qs_hbm, k_hbm, src_hbm, dst_hbm, ex_hbm,
             sbig, dbig, qbufs, kbufs, exbufs, semq, semk, semx):
    cid = lax.axis_index("c")
    sid = lax.axis_index("s")
    wid = sid * NC + cid
    n = E_PAD // (B * NC * NS)
    c0 = wid * n

    pltpu.sync_copy(src_hbm.at[pl.ds(c0, n)], sbig)
    pltpu.sync_copy(dst_hbm.at[pl.ds(c0, n)], dbig)

    zero = jnp.zeros((16,), jnp.float32)
    for exbuf in exbufs:
        for g in range(B // 16):
            rows16 = _iota16() + g * 16
            for h in range(H, HP):
                plsc.store_scatter(exbuf, [rows16, _full16(h)], zero)

    HB = B // 2

    def issue_gather(t, p):
        pltpu.async_copy(qs_hbm.at[dbig.at[t]], qbufs[p], semq[p])
        pltpu.async_copy(k_hbm.at[sbig.at[t]], kbufs[p], semk[p])

    def wait_gather(t, p):
        pltpu.make_async_copy(qs_hbm.at[dbig.at[t]], qbufs[p], semq[p]).wait()
        pltpu.make_async_copy(k_hbm.at[sbig.at[t]], kbufs[p], semk[p]).wait()

    def sub(t, p):
        @pl.when(t + 1 < n)
        def _():
            issue_gather(t + 1, 1 - p)
        wait_gather(t, p)

        @pl.when(t >= 2)
        def _():
            pltpu.make_async_copy(
                exbufs[p], ex_hbm.at[pl.ds((c0 + t - 2) * B, B)],
                semx[p]).wait()

        perm = [jnp.bitwise_and(_iota16() + s, 15) for s in range(16)]

        def group(g, _):
            rows16 = _iota16() + g * 16
            for h in range(H):
                acc = None
                for b in range(C // 16):
                    base = h * C + 16 * b
                    for s in range(16):
                        col = perm[s] + base
                        qv = plsc.load_gather(qbufs[p], [rows16, col])
                        kv = plsc.load_gather(kbufs[p], [rows16, col])
                        acc = qv * kv if acc is None else acc + qv * kv
                plsc.store_scatter(exbufs[p], [rows16, _full16(h)],
                                   jnp.exp(acc))
            return 0

        lax.fori_loop(0, B // 16, group, 0)
        pltpu.async_copy(exbufs[p], ex_hbm.at[pl.ds((c0 + t) * B, B)],
                         semx[p])

    issue_gather(0, 0)

    def body(i, _):
        sub(2 * i, 0)
        sub(2 * i + 1, 1)
        return 0

    lax.fori_loop(0, n // 2, body, 0)
    pltpu.make_async_copy(exbufs[0], ex_hbm.at[pl.ds((c0 + n - 2) * B, B)],
                          semx[0]).wait()
    pltpu.make_async_copy(exbufs[1], ex_hbm.at[pl.ds((c0 + n - 1) * B, B)],
                          semx[1]).wait()


def _edge_ex(qs, k, src2, dst2, H, C):
    HC = H * C
    n = E_PAD // (B * NC * NS)
    f = pl.kernel(
        functools.partial(_ex_body, H, C),
        out_type=jax.ShapeDtypeStruct((E_PAD, HP), jnp.float32),
        mesh=_MESH,
        compiler_params=_SC_PARAMS,
        scratch_types=[
            pltpu.VMEM((n, B), jnp.int32),
            pltpu.VMEM((n, B), jnp.int32),
            [pltpu.VMEM((B, HC), jnp.float32)] * 2,
            [pltpu.VMEM((B, HC), jnp.float32)] * 2,
            [pltpu.VMEM((B, HP), jnp.float32)] * 2,
            [pltpu.SemaphoreType.DMA] * 2,
            [pltpu.SemaphoreType.DMA] * 2,
            [pltpu.SemaphoreType.DMA] * 2,
        ],
    )
    return f(qs, k, src2, dst2)


W = 8


def _scatter_pass(H, C, h_idx, v_hbm, ex_hbm, src_hbm, dst_hbm,
                  acc_out, den_out, zacc_hbm, zden_hbm,
                  sbig, dbig, vbufs, exbufs, wbufs, accsh, densh,
                  semv, seme, semw, semd, do_den):
    sid = lax.axis_index("s")
    rows_per = N_PAD // NS
    r0 = sid * rows_per
    pltpu.sync_copy(zacc_hbm.at[pl.ds(r0, rows_per)],
                    accsh.at[pl.ds(r0, rows_per)])
    if do_den:
        pltpu.sync_copy(zden_hbm.at[pl.ds(r0, rows_per)],
                        densh.at[pl.ds(r0, rows_per)])
    plsc.subcore_barrier()

    n = E_PAD // (B * NS * 2)
    HB = B // 2

    def issue_gather(t, p, c0):
        pltpu.async_copy(v_hbm.at[sbig.at[t]], vbufs[p], semv[p])
        pltpu.async_copy(ex_hbm.at[pl.ds((c0 + t) * B, B)], exbufs[p],
                         seme[p])

    def wait_scatter(t, p):
        pltpu.make_async_copy(wbufs[p], accsh.at[dbig.at[t]], semw[p]).wait()
        if do_den:
            pltpu.make_async_copy(exbufs[p], densh.at[dbig.at[t]],
                                  semd[p]).wait()

    def sub(t, p, c0):
        @pl.when(t >= 1)
        def _():
            wait_scatter(t - 1, 1 - p)

        @pl.when(t + 1 < n)
        def _():
            issue_gather(t + 1, 1 - p, c0)

        pltpu.make_async_copy(v_hbm.at[sbig.at[t]], vbufs[p], semv[p]).wait()
        pltpu.make_async_copy(ex_hbm.at[pl.ds((c0 + t) * B, B)], exbufs[p],
                              seme[p]).wait()
        perm = [jnp.bitwise_and(_iota16() + s, W - 1) for s in range(W)]
        for g in range(B // 16):
            rows16 = _iota16() + g * 16
            exh = plsc.load_gather(exbufs[p], [rows16, _full16(h_idx)])
            for s in range(W):
                vv = plsc.load_gather(vbufs[p], [rows16, perm[s]])
                plsc.store_scatter(wbufs[p], [rows16, perm[s]], vv * exh)
        pltpu.async_copy(wbufs[p], accsh.at[dbig.at[t]], add=True,
                         sem=semw[p])
        if do_den:
            pltpu.async_copy(exbufs[p], densh.at[dbig.at[t]], add=True,
                             sem=semd[p])

    def half_body(half, _):
        c0 = sid * 2 * n + half * n
        pltpu.sync_copy(src_hbm.at[pl.ds(c0, n)], sbig)
        pltpu.sync_copy(dst_hbm.at[pl.ds(c0, n)], dbig)
        issue_gather(0, 0, c0)

        def body(i, _):
            sub(2 * i, 0, c0)
            sub(2 * i + 1, 1, c0)
            return 0

        lax.fori_loop(0, n // 2, body, 0)
        wait_scatter(n - 1, 1)
        return 0

    lax.fori_loop(0, 2, half_body, 0)

    plsc.subcore_barrier()
    pltpu.sync_copy(accsh.at[pl.ds(r0, rows_per)],
                    acc_out.at[pl.ds(r0, rows_per)])
    if do_den:
        pltpu.sync_copy(densh.at[pl.ds(r0, rows_per)],
                        den_out.at[pl.ds(r0, rows_per)])


def _scatter_body(H, C, P, *refs):
    vs = refs[:2 * P]
    ex_hbm, src_hbm, dst_hbm, zacc_hbm, zden_hbm = refs[2 * P:2 * P + 5]
    accs = refs[2 * P + 5:4 * P + 5]
    den_out = refs[4 * P + 5]
    (sbig, dbig, vbufs, exbufs, wbufs, accsh, densh,
     semv, seme, semw, semd) = refs[4 * P + 6:]
    cid = lax.axis_index("c")

    for c in range(NC):
        @pl.when(cid == c)
        def _(c=c):
            for j in range(P):
                g = 2 * j + c
                _scatter_pass(H, C, (g * W) // C, vs[g], ex_hbm,
                              src_hbm, dst_hbm, accs[g], den_out,
                              zacc_hbm, zden_hbm,
                              sbig, dbig, vbufs, exbufs, wbufs,
                              accsh, densh, semv, seme, semw, semd,
                              do_den=(c == 1 and j == 0))


def _edge_scatter(vgroups, ex, src2, dst2, H, C):
    P = len(vgroups) // 2
    n = E_PAD // (B * NS * 2)
    zacc = jnp.zeros((N_PAD, W), jnp.float32)
    zden = jnp.zeros((N_PAD, HP), jnp.float32)
    f = pl.kernel(
        functools.partial(_scatter_body, H, C, P),
        out_type=[jax.ShapeDtypeStruct((N_PAD, W), jnp.float32)] * (2 * P)
        + [jax.ShapeDtypeStruct((N_PAD, HP), jnp.float32)],
        mesh=_MESH,
        compiler_params=_SC_PARAMS,
        scratch_types=[
            pltpu.VMEM((n, B), jnp.int32),
            pltpu.VMEM((n, B), jnp.int32),
            [pltpu.VMEM((B, W), jnp.float32)] * 2,
            [pltpu.VMEM((B, HP), jnp.float32)] * 2,
            [pltpu.VMEM((B, W), jnp.float32)] * 2,
            pltpu.VMEM_SHARED((N_PAD, W), jnp.float32),
            pltpu.VMEM_SHARED((N_PAD, HP), jnp.float32),
            [pltpu.SemaphoreType.DMA] * 2,
            [pltpu.SemaphoreType.DMA] * 2,
            [pltpu.SemaphoreType.DMA] * 2,
            [pltpu.SemaphoreType.DMA] * 2,
        ],
    )
    outs = f(*vgroups, ex, src2, dst2, zacc, zden)
    return outs[:-1], outs[-1]


def _finalize_h(Hp, Cp, NGp, acc_refs, den_ref, hws_ref):
    acc = jnp.concatenate([r[...] for r in acc_refs], axis=1)
    dn = den_ref[...]
    R = acc.shape[0]
    denb = jnp.concatenate(
        [jnp.broadcast_to(dn[:, h:h + 1], (R, Cp)) for h in range(Hp)],
        axis=1)
    return acc / (denb + 1e-16) + hws_ref[...]


def _proj_body(C, NG, prev, *refs):
    if prev is None:
        h_ref = refs[0]
        refs = refs[1:]
        h = h_ref[...]
    else:
        Hp, Cp, NGp = prev
        acc_refs = refs[:NGp]
        den_ref = refs[NGp]
        hws_ref = refs[NGp + 1]
        refs = refs[NGp + 2:]
        h = _finalize_h(Hp, Cp, NGp, acc_refs, den_ref, hws_ref)
    (wq, bq, wk, bk, wv, bv, ws, bs,
     qs_ref, k_ref, hws_ref, *v_refs) = refs
    dot = lambda a, b: lax.dot_general(a, b, (((1,), (0,)), ((), ())),
                                       preferred_element_type=jnp.float32)
    qs_ref[...] = (dot(h, wq[...]) + bq[...]) * (1.0 / math.sqrt(C))
    k_ref[...] = dot(h, wk[...]) + bk[...]
    v = dot(h, wv[...]) + bv[...]
    for g in range(NG):
        v_refs[g][...] = v[:, g * W:(g + 1) * W]
    hws_ref[...] = dot(h, ws[...]) + bs[...]


def _projections(hin, Wq, bq, Wk, bk, Wv, bv, Ws, bs, C, prev=None):
    HC = Wq.shape[1]
    NG = HC // W
    R = 2176
    grid = N_PAD // R
    row_spec = lambda w: pl.BlockSpec((R, w), lambda i: (i, 0))
    full = lambda a: pl.BlockSpec(a.shape, lambda i: (0, 0))
    out_shapes = (
        [jax.ShapeDtypeStruct((N_PAD, HC), jnp.float32)] * 3
        + [jax.ShapeDtypeStruct((N_PAD, W), jnp.float32)] * NG
    )
    if prev is None:
        head_args = [hin]
        head_specs = [row_spec(hin.shape[1])]
    else:
        accs, den, hws = hin
        head_args = list(accs) + [den, hws]
        head_specs = ([row_spec(W)] * len(accs)
                      + [row_spec(HP), row_spec(hws.shape[1])])
    warg = (Wq, bq[None, :], Wk, bk[None, :], Wv, bv[None, :],
            Ws, bs[None, :])
    outs = pl.pallas_call(
        functools.partial(_proj_body, C, NG,
                          None if prev is None else prev),
        grid=(grid,),
        in_specs=head_specs + [full(a) for a in warg],
        out_specs=[row_spec(HC)] * 3 + [row_spec(W)] * NG,
        out_shape=out_shapes,
    )(*head_args, *warg)
    return outs[0], outs[1], outs[2], outs[3:]


def _final_body(H, C, NG, *refs):
    acc_refs = refs[:NG]
    den_ref, hws_ref, out_ref = refs[NG:]
    acc = jnp.concatenate([r[...] for r in acc_refs], axis=1)
    dn = den_ref[...]
    R = acc.shape[0]
    denb = jnp.concatenate(
        [jnp.broadcast_to(dn[:, h:h + 1], (R, C)) for h in range(H)], axis=1)
    out_ref[...] = acc / (denb + 1e-16) + hws_ref[...]


def _finalize(accs, den, hws, H, C):
    HC = H * C
    NG = len(accs)
    R = 2176
    grid = N_PAD // R
    spec = lambda w: pl.BlockSpec((R, w), lambda i: (i, 0))
    return pl.pallas_call(
        functools.partial(_final_body, H, C, NG),
        grid=(grid,),
        in_specs=[spec(W)] * NG + [spec(HP), spec(HC)],
        out_specs=spec(HC),
        out_shape=jax.ShapeDtypeStruct((N_PAD, HC), jnp.float32),
    )(*accs, den, hws)


def _tail_body(Hp, Cp, NGp, *refs):
    acc_refs = refs[:NGp]
    den_ref, hws_ref, b_ref, demo_ref, wc1, bc1, wc2, bc2, out_ref, gs_ref = \
        refs[NGp:]
    i = pl.program_id(0)
    h = _finalize_h(Hp, Cp, NGp, acc_refs, den_ref, hws_ref)
    R = h.shape[0]
    b = b_ref[...].reshape(R)
    oh = (b[:, None] == lax.broadcasted_iota(jnp.int32, (1, G), 1)
          ).astype(jnp.float32)
    hext = jnp.concatenate([h, jnp.ones((R, 1), jnp.float32)], axis=1)
    part = lax.dot_general(oh, hext, (((0,), (0,)), ((), ())),
                           preferred_element_type=jnp.float32)

    @pl.when(i == 0)
    def _():
        gs_ref[...] = jnp.zeros_like(gs_ref)

    gs_ref[...] += part

    @pl.when(i == pl.num_programs(0) - 1)
    def _():
        gs = gs_ref[...]
        D = gs.shape[1] - 1
        gmean = gs[:, :D] / jnp.maximum(gs[:, D:], 1.0)
        feat = jnp.concatenate([gmean, demo_ref[...]], axis=1)
        dot = lambda a, b: lax.dot_general(a, b, (((1,), (0,)), ((), ())),
                                           preferred_element_type=jnp.float32)
        hh = jnp.maximum(dot(feat, wc1[...]) + bc1[...], 0.0)
        out_ref[...] = dot(hh, wc2[...]) + bc2[...]


def _tail(accs, den, hws, batch3d, demo, Wc1, bc1, Wc2, bc2, Hp, Cp):
    NGp = len(accs)
    D = Hp * Cp
    R = 2176
    grid = N_PAD // R
    row_spec = lambda w: pl.BlockSpec((R, w), lambda i: (i, 0))
    full = lambda a: pl.BlockSpec(a.shape, lambda i: tuple(0 for _ in a.shape))
    args = (*accs, den, hws, batch3d, demo, Wc1, bc1[None, :],
            Wc2, bc2[None, :])
    return pl.pallas_call(
        functools.partial(_tail_body, Hp, Cp, NGp),
        grid=(grid,),
        in_specs=[row_spec(W)] * NGp
        + [row_spec(HP), row_spec(D),
           pl.BlockSpec((1, 1, 2176), lambda i: (i, 0, 0)),
           full(demo), full(Wc1), full(bc1[None, :]),
           full(Wc2), full(bc2[None, :])],
        out_specs=pl.BlockSpec((G, Wc2.shape[1]), lambda i: (0, 0)),
        out_shape=jax.ShapeDtypeStruct((G, Wc2.shape[1]), jnp.float32),
        scratch_shapes=[pltpu.VMEM((G, D + 1), jnp.float32)],
    )(*args)


def kernel(x, edge_index, batch, demographics, emb,
           Wq1, bq1, Wk1, bk1, Wv1, bv1, Ws1, bs1,
           Wq2, bq2, Wk2, bk2, Wv2, bv2, Ws2, bs2,
           Wq3, bq3, Wk3, bk3, Wv3, bv3, Ws3, bs3,
           Wc1, bc1, Wc2, bc2):
    src = edge_index[0]
    dst = edge_index[1]

    pad_e = E_PAD - E
    srcp = jnp.concatenate(
        [src, (jnp.arange(pad_e, dtype=jnp.int32) * 97) % N]
    ).reshape(E_PAD // B, B)
    dstp = jnp.concatenate(
        [dst, N + (jnp.arange(pad_e, dtype=jnp.int32) % (N_PAD - N))]
    ).reshape(E_PAD // B, B)

    x_pad = jnp.concatenate(
        [x, jnp.arange(X_PAD - N, dtype=jnp.int32) % VOCAB])

    h0 = _emb_lookup(emb, x_pad)

    qs, k, hws1, vg = _projections(h0, Wq1, bq1, Wk1, bk1, Wv1, bv1,
                                   Ws1, bs1, 16)
    ex = _edge_ex(qs, k, srcp, dstp, 3, 16)
    accs1, den1 = _edge_scatter(vg, ex, srcp, dstp, 3, 16)

    qs, k, hws2, vg = _projections((accs1, den1, hws1), Wq2, bq2, Wk2, bk2,
                                   Wv2, bv2, Ws2, bs2, 48, prev=(3, 16, 6))
    ex = _edge_ex(qs, k, srcp, dstp, 1, 48)
    accs2, den2 = _edge_scatter(vg, ex, srcp, dstp, 1, 48)

    qs, k, hws3, vg = _projections((accs2, den2, hws2), Wq3, bq3, Wk3, bk3,
                                   Wv3, bv3, Ws3, bs3, 32, prev=(1, 48, 6))
    ex = _edge_ex(qs, k, srcp, dstp, 1, 32)
    accs3, den3 = _edge_scatter(vg, ex, srcp, dstp, 1, 32)

    batchp = jnp.concatenate(
        [batch, jnp.full((N_PAD - N,), G, jnp.int32)]).reshape(23, 1, 2176)
    return _tail(accs3, den3, hws3, batchp, demographics,
                 Wc1, bc1, Wc2, bc2, 1, 32)
